# baseline (speedup 1.0000x reference)
import jax
import jax.numpy as jnp
from jax.experimental import pallas as pl
from jax.experimental.pallas import tpu as pltpu

N = 4096
T = 4
D = 128
H = 64
C = 16

BLK1 = 256
NB1 = N // BLK1
BLK2 = 256
NB2 = N // (BLK2 * T)


def _fused_kernel(c_ref, x_ref, w1_ref, b1_ref, w2_ref, b2_ref, adj_ref,
                  out_ref, a_ref, h_ref, acc_ref, xw1_ref, hw2_ref):
    i = pl.program_id(0)
    t = pl.program_id(1)

    @pl.when(jnp.logical_and(i == 0, t == 0))
    def _():
        xw1_ref[...] = jnp.dot(x_ref[...], w1_ref[...],
                               preferred_element_type=jnp.float32
                               ).astype(jnp.bfloat16)

    @pl.when(i < NB1)
    def _():
        blk = c_ref[t] * adj_ref[...]

        @pl.when(t == 0)
        def _():
            acc_ref[...] = blk

        @pl.when(jnp.logical_and(t > 0, t < T - 1))
        def _():
            acc_ref[...] += blk

        @pl.when(t == T - 1)
        def _():
            a_bf = (acc_ref[...] + blk).astype(jnp.bfloat16)
            a_ref[pl.ds(i * BLK1, BLK1), :] = a_bf
            h_ref[pl.ds(i * BLK1, BLK1), :] = jax.nn.relu(
                jnp.dot(a_bf, xw1_ref[...],
                        preferred_element_type=jnp.float32) + b1_ref[...]
            ).astype(jnp.bfloat16)

    @pl.when(jnp.logical_and(i == NB1, t == 0))
    def _():
        hw2_ref[...] = jnp.dot(h_ref[...], w2_ref[...],
                               preferred_element_type=jnp.float32
                               ).astype(jnp.bfloat16)

    @pl.when(i >= NB1)
    def _():
        j = (i - NB1) * T + t
        logits = jnp.dot(a_ref[pl.ds(j * BLK2, BLK2), :], hw2_ref[...],
                         preferred_element_type=jnp.float32) + b2_ref[...]
        m = jnp.max(logits, axis=-1, keepdims=True)
        e = jnp.exp(logits - m)
        out_ref[...] = e / jnp.sum(e, axis=-1, keepdims=True)


@jax.jit
def kernel(feats, adj, lam, W1, b1, W2, b2):
    x = feats[:, -1, :]
    one_m = 1.0 - lam
    c = jnp.stack([one_m ** (T - 1)]
                  + [lam * one_m ** (T - 1 - t) for t in range(1, T)])
    c = c.astype(jnp.float32)
    adj_flat = adj.reshape(T * N, N)

    def adj_map(i, t):
        ic = jnp.minimum(i, NB1 - 1)
        return (t * NB1 + ic, 0)

    def out_map(i, t):
        j = jnp.maximum(i - NB1, 0)
        return (j * T + t, 0)

    out = pl.pallas_call(
        _fused_kernel,
        grid=(NB1 + NB2, T),
        in_specs=[
            pl.BlockSpec(memory_space=pltpu.SMEM),
            pl.BlockSpec((N, D), lambda i, t: (0, 0)),
            pl.BlockSpec((D, H), lambda i, t: (0, 0)),
            pl.BlockSpec((1, H), lambda i, t: (0, 0)),
            pl.BlockSpec((H, C), lambda i, t: (0, 0)),
            pl.BlockSpec((1, C), lambda i, t: (0, 0)),
            pl.BlockSpec((BLK1, N), adj_map),
        ],
        out_specs=pl.BlockSpec((BLK2, C), out_map),
        out_shape=jax.ShapeDtypeStruct((N, C), jnp.float32),
        scratch_shapes=[
            pltpu.VMEM((N, N), jnp.bfloat16),
            pltpu.VMEM((N, H), jnp.bfloat16),
            pltpu.VMEM((BLK1, N), jnp.float32),
            pltpu.VMEM((N, H), jnp.bfloat16),
            pltpu.VMEM((N, C), jnp.bfloat16),
        ],
    )(c, x, W1, b1.reshape(1, H), W2, b2.reshape(1, C), adj_flat)

    return out


# speedup vs baseline: 1.4167x; 1.4167x over previous
import jax
import jax.numpy as jnp
from jax.experimental import pallas as pl
from jax.experimental.pallas import tpu as pltpu

N = 4096
T = 4
D = 128
H = 64
C = 16

BLK1 = 128
BLK2 = 1024
NB1 = N // BLK1
NB2 = N // BLK2


def _fused_kernel(c_ref, x_ref, w1_ref, b1_ref, w2_ref, b2_ref,
                  adj0_ref, adj1_ref, adj2_ref, adj3_ref,
                  out_ref, a_ref, h_ref, xw1_ref, hw2_ref):
    i = pl.program_id(0)

    @pl.when(i == 0)
    def _():
        xw1_ref[...] = jnp.dot(x_ref[...], w1_ref[...],
                               preferred_element_type=jnp.float32
                               ).astype(jnp.bfloat16)

    @pl.when(i < NB1)
    def _():
        a_blk = (c_ref[0] * adj0_ref[0] + c_ref[1] * adj1_ref[0]
                 + c_ref[2] * adj2_ref[0] + c_ref[3] * adj3_ref[0])
        a_bf = a_blk.astype(jnp.bfloat16)
        a_ref[pl.ds(i * BLK1, BLK1), :] = a_bf
        h_ref[pl.ds(i * BLK1, BLK1), :] = jax.nn.relu(
            jnp.dot(a_bf, xw1_ref[...],
                    preferred_element_type=jnp.float32) + b1_ref[...]
        ).astype(jnp.bfloat16)

    @pl.when(i == NB1)
    def _():
        hw2_ref[...] = jnp.dot(h_ref[...], w2_ref[...],
                               preferred_element_type=jnp.float32
                               ).astype(jnp.bfloat16)

    @pl.when(i >= NB1)
    def _():
        j = i - NB1
        logits = jnp.dot(a_ref[pl.ds(j * BLK2, BLK2), :], hw2_ref[...],
                         preferred_element_type=jnp.float32) + b2_ref[...]
        m = jnp.max(logits, axis=-1, keepdims=True)
        e = jnp.exp(logits - m)
        out_ref[...] = e / jnp.sum(e, axis=-1, keepdims=True)


@jax.jit
def kernel(feats, adj, lam, W1, b1, W2, b2):
    x = feats[:, -1, :]
    one_m = 1.0 - lam
    c = jnp.stack([one_m ** (T - 1)]
                  + [lam * one_m ** (T - 1 - t) for t in range(1, T)])
    c = c.astype(jnp.float32)

    def adj_spec(t):
        return pl.BlockSpec((1, BLK1, N),
                            lambda i, _t=t: (_t, jnp.minimum(i, NB1 - 1), 0))

    out = pl.pallas_call(
        _fused_kernel,
        grid=(NB1 + NB2,),
        in_specs=[
            pl.BlockSpec(memory_space=pltpu.SMEM),
            pl.BlockSpec((N, D), lambda i: (0, 0)),
            pl.BlockSpec((D, H), lambda i: (0, 0)),
            pl.BlockSpec((1, H), lambda i: (0, 0)),
            pl.BlockSpec((H, C), lambda i: (0, 0)),
            pl.BlockSpec((1, C), lambda i: (0, 0)),
            adj_spec(0), adj_spec(1), adj_spec(2), adj_spec(3),
        ],
        out_specs=pl.BlockSpec((BLK2, C),
                               lambda i: (jnp.maximum(i - NB1, 0), 0)),
        out_shape=jax.ShapeDtypeStruct((N, C), jnp.float32),
        scratch_shapes=[
            pltpu.VMEM((N, N), jnp.bfloat16),
            pltpu.VMEM((N, H), jnp.bfloat16),
            pltpu.VMEM((N, H), jnp.bfloat16),
            pltpu.VMEM((N, C), jnp.bfloat16),
        ],
    )(c, x, W1, b1.reshape(1, H), W2, b2.reshape(1, C), adj, adj, adj, adj)

    return out


# speedup vs baseline: 1.4198x; 1.0022x over previous
import jax
import jax.numpy as jnp
from jax.experimental import pallas as pl
from jax.experimental.pallas import tpu as pltpu

N = 4096
T = 4
D = 128
H = 64
C = 16

BLK = 128
NB = N // BLK
GROUPS = 2
S = GROUPS * T
BLK2 = 1024
NB2 = N // BLK2


def _fused_kernel(c_ref, x_ref, w1_ref, b1_ref, w2_ref, b2_ref, adj_ref,
                  out_ref, a_ref, h_ref, xw1_ref, hw2_ref, slots_ref, sems):

    def start_group(i):
        g = (i % GROUPS) * T
        for t in range(T):
            pltpu.make_async_copy(
                adj_ref.at[t, pl.ds(i * BLK, BLK), :],
                slots_ref.at[g + t],
                sems.at[g + t],
            ).start()

    for i in range(GROUPS):
        start_group(i)

    xw1_ref[...] = jnp.dot(x_ref[...], w1_ref[...],
                           preferred_element_type=jnp.float32
                           ).astype(jnp.bfloat16)

    def body(i, carry):
        g = (i % GROUPS) * T
        for t in range(T):
            pltpu.make_async_copy(
                adj_ref.at[t, pl.ds(i * BLK, BLK), :],
                slots_ref.at[g + t],
                sems.at[g + t],
            ).wait()
        a_blk = (c_ref[0] * slots_ref[g] + c_ref[1] * slots_ref[g + 1]
                 + c_ref[2] * slots_ref[g + 2] + c_ref[3] * slots_ref[g + 3])
        a_bf = a_blk.astype(jnp.bfloat16)
        a_ref[pl.ds(i * BLK, BLK), :] = a_bf
        h_ref[pl.ds(i * BLK, BLK), :] = jax.nn.relu(
            jnp.dot(a_bf, xw1_ref[...],
                    preferred_element_type=jnp.float32) + b1_ref[...]
        ).astype(jnp.bfloat16)

        @pl.when(i + GROUPS < NB)
        def _():
            start_group(i + GROUPS)

        return carry

    jax.lax.fori_loop(0, NB, body, 0)

    hw2_ref[...] = jnp.dot(h_ref[...], w2_ref[...],
                           preferred_element_type=jnp.float32
                           ).astype(jnp.bfloat16)

    def body2(j, carry):
        logits = jnp.dot(a_ref[pl.ds(j * BLK2, BLK2), :], hw2_ref[...],
                         preferred_element_type=jnp.float32) + b2_ref[...]
        m = jnp.max(logits, axis=-1, keepdims=True)
        e = jnp.exp(logits - m)
        out_ref[pl.ds(j * BLK2, BLK2), :] = e / jnp.sum(e, axis=-1,
                                                        keepdims=True)
        return carry

    jax.lax.fori_loop(0, NB2, body2, 0)


@jax.jit
def kernel(feats, adj, lam, W1, b1, W2, b2):
    x = feats[:, -1, :]
    one_m = 1.0 - lam
    c = jnp.stack([one_m ** (T - 1)]
                  + [lam * one_m ** (T - 1 - t) for t in range(1, T)])
    c = c.astype(jnp.float32)

    out = pl.pallas_call(
        _fused_kernel,
        in_specs=[
            pl.BlockSpec(memory_space=pltpu.SMEM),
            pl.BlockSpec(memory_space=pltpu.VMEM),
            pl.BlockSpec(memory_space=pltpu.VMEM),
            pl.BlockSpec(memory_space=pltpu.VMEM),
            pl.BlockSpec(memory_space=pltpu.VMEM),
            pl.BlockSpec(memory_space=pltpu.VMEM),
            pl.BlockSpec(memory_space=pl.ANY),
        ],
        out_specs=pl.BlockSpec(memory_space=pltpu.VMEM),
        out_shape=jax.ShapeDtypeStruct((N, C), jnp.float32),
        scratch_shapes=[
            pltpu.VMEM((N, N), jnp.bfloat16),
            pltpu.VMEM((N, H), jnp.bfloat16),
            pltpu.VMEM((N, H), jnp.bfloat16),
            pltpu.VMEM((N, C), jnp.bfloat16),
            pltpu.VMEM((S, BLK, N), jnp.float32),
            pltpu.SemaphoreType.DMA((S,)),
        ],
    )(c, x, W1, b1.reshape(1, H), W2, b2.reshape(1, C), adj)

    return out
